# baseline (speedup 1.0000x reference)
import functools

import jax
import jax.numpy as jnp
from jax import lax
from jax.experimental import pallas as pl
from jax.experimental.pallas import tpu as pltpu
from jax.experimental.pallas import tpu_sc as plsc

BATCH = 16384
EMBED_DIM = 32
NUM_CORES = 2
NUM_SUBCORES = 16
NUM_WORKERS = NUM_CORES * NUM_SUBCORES
ROWS_PER_WORKER = BATCH // NUM_WORKERS
CHUNK = 128
NUM_CHUNKS = ROWS_PER_WORKER // CHUNK
LANES = 16


def _fm_body(uid_hbm, iid_hbm, ut_hbm, it_hbm, out_hbm,
             uidx_v, iidx_v, urows_v, irows_v, out_v, sem):
    wid = lax.axis_index("s") * NUM_CORES + lax.axis_index("c")
    base = wid * NUM_CHUNKS

    pltpu.sync_copy(uid_hbm.at[pl.ds(base, NUM_CHUNKS)], uidx_v)
    pltpu.sync_copy(iid_hbm.at[pl.ds(base, NUM_CHUNKS)], iidx_v)

    copies = []
    for j in range(NUM_CHUNKS):
        dst = urows_v.at[pl.ds(j * CHUNK, CHUNK)]
        copies.append(pltpu.async_copy(ut_hbm.at[uidx_v.at[j]], dst, sem))
        dst = irows_v.at[pl.ds(j * CHUNK, CHUNK)]
        copies.append(pltpu.async_copy(it_hbm.at[iidx_v.at[j]], dst, sem))
    for c in copies:
        c.wait()

    lane = lax.iota(jnp.int32, LANES)

    def group(g, carry):
        acc = jnp.zeros((LANES,), jnp.float32)
        base_r = g * LANES
        for j in range(LANES):
            r = base_r + j
            u0 = urows_v[r, pl.ds(0, LANES)]
            u1 = urows_v[r, pl.ds(LANES, LANES)]
            i0 = irows_v[r, pl.ds(0, LANES)]
            i1 = irows_v[r, pl.ds(LANES, LANES)]
            s = (u0 + i0 + u0 * i0) + (u1 + i1 + u1 * i1)
            acc = jnp.where(lane == j, jnp.sum(s), acc)
        out_v[pl.ds(g * LANES, LANES)] = acc
        return carry

    lax.fori_loop(0, ROWS_PER_WORKER // LANES, group, 0)
    pltpu.sync_copy(out_v, out_hbm.at[pl.ds(wid * ROWS_PER_WORKER,
                                            ROWS_PER_WORKER)])


def kernel(user_ids, item_ids, user_table, item_table):
    uids = user_ids.astype(jnp.int32).reshape(NUM_WORKERS * NUM_CHUNKS, CHUNK)
    iids = item_ids.astype(jnp.int32).reshape(NUM_WORKERS * NUM_CHUNKS, CHUNK)
    mesh = plsc.VectorSubcoreMesh(core_axis_name="c", subcore_axis_name="s")
    fm = functools.partial(
        pl.kernel,
        mesh=mesh,
        compiler_params=pltpu.CompilerParams(needs_layout_passes=False,
                                             use_tc_tiling_on_sc=False),
        out_type=jax.ShapeDtypeStruct((BATCH,), jnp.float32),
        scratch_types=[
            pltpu.VMEM((NUM_CHUNKS, CHUNK), jnp.int32),
            pltpu.VMEM((NUM_CHUNKS, CHUNK), jnp.int32),
            pltpu.VMEM((ROWS_PER_WORKER, EMBED_DIM), jnp.float32),
            pltpu.VMEM((ROWS_PER_WORKER, EMBED_DIM), jnp.float32),
            pltpu.VMEM((ROWS_PER_WORKER,), jnp.float32),
            pltpu.SemaphoreType.DMA,
        ],
    )(_fm_body)
    out = fm(uids, iids, user_table, item_table)
    return out.reshape(BATCH, 1)
